# Initial kernel scaffold; baseline (speedup 1.0000x reference)
import jax
import jax.numpy as jnp
from jax.experimental import pallas as pl


def kernel(x, edge_index, W1, theta1, W2, theta2):
    raise NotImplementedError("write your pallas kernel here")


# speedup vs baseline: 7.3087x; 7.3087x over previous
import functools

import jax
import jax.numpy as jnp
from jax import lax
from jax.experimental import pallas as pl
from jax.experimental.pallas import tpu as pltpu
from jax.experimental.pallas import tpu_sc as plsc

N = 10000
D = 128
HID = 64
HEADS = 4
CLS = 7
K = 16

NPAD = 10240
EPAD = 327680
CPAD = 16
TILES = 16
NODES_PT = NPAD // TILES
ECHUNK = 128
ROWS_PT = EPAD // ECHUNK // TILES
NBLK = 128
NBLKS_PT = NODES_PT // NBLK
NB = 4


def _splat(v16, i):
  idx = jnp.full((16,), i, jnp.int32)
  return v16.at[idx].get(mode="promise_in_bounds")


def _edge_phase(src_v, dst_v, tprime_h, accum_s, buf_v, gsem, ssem):
  def _gather(c):
    s = c % NB
    return pltpu.make_async_copy(
        tprime_h.at[src_v.at[c]], buf_v.at[s], gsem.at[s])

  def _scatter(c):
    s = c % NB
    return pltpu.make_async_copy(
        buf_v.at[s], accum_s.at[dst_v.at[c]], ssem.at[s])

  _gather(0).start()
  _gather(1).start()

  def body(c, carry):
    _gather(c).wait()
    _scatter(c).start(add=True)

    @pl.when(c + 2 < ROWS_PT)
    def _():
      @pl.when(c >= 2)
      def _():
        _scatter(c - 2).wait()
      _gather(c + 2).start()
    return carry
  lax.fori_loop(0, ROWS_PT, body, 0)

  def drain(x, carry):
    _scatter(x).wait()
    return carry
  lax.fori_loop(ROWS_PT - 4, ROWS_PT, drain, 0)


def _sc_degrees(src2d, dst2d):
  def body(src_h, dst_h, deg_h, degs_s, degd_s, src_v, dst_v, ones_v, dv_v):
    cid = lax.axis_index("c")
    t = lax.axis_index("s")

    @pl.when(cid == 0)
    def _core0():
      nbase = t * NODES_PT
      pltpu.sync_copy(src_h.at[pl.ds(t * ROWS_PT, ROWS_PT)], src_v)
      pltpu.sync_copy(dst_h.at[pl.ds(t * ROWS_PT, ROWS_PT)], dst_v)

      def fill_ones(j, c):
        ones_v[pl.ds(j * 16, 16)] = jnp.full((16,), 1.0, jnp.float32)
        return c
      lax.fori_loop(0, ECHUNK // 16, fill_ones, 0)

      def fill_dv(j, c):
        dv_v[pl.ds(j * 16, 16)] = jnp.zeros((16,), jnp.float32)
        return c
      lax.fori_loop(0, NODES_PT // 16, fill_dv, 0)

      pltpu.sync_copy(dv_v, degs_s.at[pl.ds(nbase, NODES_PT)])
      pltpu.sync_copy(dv_v, degd_s.at[pl.ds(nbase, NODES_PT)])
      plsc.subcore_barrier()

      def degbody(c, carry):
        pltpu.sync_copy(ones_v, degs_s.at[src_v.at[c]], add=True)
        pltpu.sync_copy(ones_v, degd_s.at[dst_v.at[c]], add=True)
        return carry
      lax.fori_loop(0, ROWS_PT, degbody, 0)

      plsc.subcore_barrier()
      pltpu.sync_copy(degs_s.at[pl.ds(nbase, NODES_PT)],
                      deg_h.at[0, pl.ds(nbase, NODES_PT)])
      pltpu.sync_copy(degd_s.at[pl.ds(nbase, NODES_PT)],
                      deg_h.at[1, pl.ds(nbase, NODES_PT)])

  mesh = plsc.VectorSubcoreMesh(core_axis_name="c", subcore_axis_name="s")
  f = pl.kernel(
      body,
      out_type=[jax.ShapeDtypeStruct((2, NPAD), jnp.float32)],
      mesh=mesh,
      compiler_params=pltpu.CompilerParams(use_tc_tiling_on_sc=False),
      scratch_types=[
          pltpu.VMEM_SHARED((NPAD,), jnp.float32),
          pltpu.VMEM_SHARED((NPAD,), jnp.float32),
          pltpu.VMEM((ROWS_PT, ECHUNK), jnp.int32),
          pltpu.VMEM((ROWS_PT, ECHUNK), jnp.int32),
          pltpu.VMEM((ECHUNK,), jnp.float32),
          pltpu.VMEM((NODES_PT,), jnp.float32),
      ],
  )
  return f(src2d, dst2d)[0]


def _tc_fg(deg):
  def body(d_ref, o_ref):
    o_ref[...] = lax.rsqrt(jnp.maximum(d_ref[...], 1.0))

  return pl.pallas_call(
      body,
      out_shape=jax.ShapeDtypeStruct((2, NPAD), jnp.float32),
  )(deg)


def _recurrence(fdim, with_out):
  def body(src_h, dst_h, x0_h, fg_h, th_h, out_h, tstack_h, tprime_h,
           accum_s, src_v, dst_v, f_v, g_v, th_v, buf_v, gsem, ssem):
    cid = lax.axis_index("c")
    t = lax.axis_index("s")

    @pl.when(cid == 0)
    def _core0():
      nbase = t * NODES_PT
      pltpu.sync_copy(src_h.at[pl.ds(t * ROWS_PT, ROWS_PT)], src_v)
      pltpu.sync_copy(dst_h.at[pl.ds(t * ROWS_PT, ROWS_PT)], dst_v)
      pltpu.sync_copy(fg_h.at[0, pl.ds(nbase, NODES_PT)], f_v)
      pltpu.sync_copy(fg_h.at[1, pl.ds(nbase, NODES_PT)], g_v)
      pltpu.sync_copy(th_h.at[0], th_v)
      th16 = th_v[...]
      th0 = _splat(th16, 0)

      def _sl(b):
        return pl.ds(nbase + b * NBLK, NBLK)

      def fill_zero(r, c):
        for fb in range(fdim // 16):
          buf_v[2, r, pl.ds(fb * 16, 16)] = jnp.zeros((16,), jnp.float32)
        return c
      lax.fori_loop(0, NBLK, fill_zero, 0)

      def zacc(b, c):
        pltpu.sync_copy(buf_v.at[2], accum_s.at[_sl(b)])
        return c
      lax.fori_loop(0, NBLKS_PT, zacc, 0)

      def initb(b, c):
        sl = _sl(b)
        pltpu.sync_copy(x0_h.at[sl], buf_v.at[0])
        pltpu.sync_copy(buf_v.at[0], tstack_h.at[0, sl])

        def grp(jj, c2):
          f16 = f_v[pl.ds(b * NBLK + jj * 16, 16)]

          def lane(i2, c3):
            fs = _splat(f16, i2)
            r = jj * 16 + i2
            for fb in range(fdim // 16):
              row = buf_v[0, r, pl.ds(fb * 16, 16)]
              buf_v[1, r, pl.ds(fb * 16, 16)] = fs * row
              if with_out:
                buf_v[3, r, pl.ds(fb * 16, 16)] = th0 * row
            return c3
          lax.fori_loop(0, 16, lane, 0)
          return c2
        lax.fori_loop(0, NBLK // 16, grp, 0)
        pltpu.sync_copy(buf_v.at[1], tprime_h.at[sl])
        if with_out:
          pltpu.sync_copy(buf_v.at[3], out_h.at[sl])
        return c
      lax.fori_loop(0, NBLKS_PT, initb, 0)

      plsc.subcore_barrier()

      def kstep(k, carry):
        _edge_phase(src_v, dst_v, tprime_h, accum_s, buf_v, gsem, ssem)
        plsc.subcore_barrier()

        kidx = jnp.maximum(k - 2, 0)
        sp = jnp.where(k >= 2, 1.0, 0.0).astype(jnp.float32)
        ca = jnp.where(k >= 2, -2.0, -1.0).astype(jnp.float32)
        sp16 = jnp.full((16,), 1.0, jnp.float32) * sp
        thk = _splat(th16, k)

        lax.fori_loop(0, NBLK, fill_zero, 0)

        def nodeb(b, c):
          sl = _sl(b)
          pltpu.sync_copy(accum_s.at[sl], buf_v.at[0])
          pltpu.sync_copy(buf_v.at[2], accum_s.at[sl])
          pltpu.sync_copy(tstack_h.at[kidx, sl], buf_v.at[1])
          if with_out:
            pltpu.sync_copy(out_h.at[sl], buf_v.at[3])

          def grp(jj, c2):
            g16 = g_v[pl.ds(b * NBLK + jj * 16, 16)] * ca
            f16 = f_v[pl.ds(b * NBLK + jj * 16, 16)]

            def lane(i2, c3):
              cg = _splat(g16, i2)
              fs = _splat(f16, i2)
              r = jj * 16 + i2
              for fb in range(fdim // 16):
                a = buf_v[0, r, pl.ds(fb * 16, 16)]
                tp = buf_v[1, r, pl.ds(fb * 16, 16)]
                tn = cg * a - sp16 * tp
                buf_v[1, r, pl.ds(fb * 16, 16)] = tn
                buf_v[0, r, pl.ds(fb * 16, 16)] = fs * tn
                if with_out:
                  buf_v[3, r, pl.ds(fb * 16, 16)] = (
                      buf_v[3, r, pl.ds(fb * 16, 16)] + thk * tn)
              return c3
            lax.fori_loop(0, 16, lane, 0)
            return c2
          lax.fori_loop(0, NBLK // 16, grp, 0)

          pltpu.sync_copy(buf_v.at[1], tstack_h.at[k, sl])
          pltpu.sync_copy(buf_v.at[0], tprime_h.at[sl])
          if with_out:
            pltpu.sync_copy(buf_v.at[3], out_h.at[sl])
          return c
        lax.fori_loop(0, NBLKS_PT, nodeb, 0)
        plsc.subcore_barrier()
        return carry

      lax.fori_loop(1, K, kstep, 0)

  mesh = plsc.VectorSubcoreMesh(core_axis_name="c", subcore_axis_name="s")
  return pl.kernel(
      body,
      out_type=[
          jax.ShapeDtypeStruct((NPAD, fdim), jnp.float32),
          jax.ShapeDtypeStruct((K, NPAD, fdim), jnp.float32),
          jax.ShapeDtypeStruct((NPAD, fdim), jnp.float32),
      ],
      mesh=mesh,
      compiler_params=pltpu.CompilerParams(use_tc_tiling_on_sc=False),
      scratch_types=[
          pltpu.VMEM_SHARED((NPAD, fdim), jnp.float32),
          pltpu.VMEM((ROWS_PT, ECHUNK), jnp.int32),
          pltpu.VMEM((ROWS_PT, ECHUNK), jnp.int32),
          pltpu.VMEM((NODES_PT,), jnp.float32),
          pltpu.VMEM((NODES_PT,), jnp.float32),
          pltpu.VMEM((16,), jnp.float32),
          pltpu.VMEM((NB, ECHUNK, fdim), jnp.float32),
          pltpu.SemaphoreType.DMA((NB,)),
          pltpu.SemaphoreType.DMA((NB,)),
      ],
  )


def _tc_matmul(x, w):
  blk = 1024

  def body(x_ref, w_ref, o_ref):
    o_ref[...] = jnp.dot(x_ref[...], w_ref[...],
                         preferred_element_type=jnp.float32)

  return pl.pallas_call(
      body,
      grid=(NPAD // blk,),
      in_specs=[
          pl.BlockSpec((blk, D), lambda i: (i, 0)),
          pl.BlockSpec((D, HID), lambda i: (0, 0)),
      ],
      out_specs=pl.BlockSpec((blk, HID), lambda i: (i, 0)),
      out_shape=jax.ShapeDtypeStruct((NPAD, HID), jnp.float32),
  )(x, w)


def _tc_combine(tstack, v):
  blk = 1024

  def body(t_ref, v_ref, o_ref):
    acc = jnp.zeros((blk, CPAD), jnp.float32)
    for k in range(K):
      acc = acc + jnp.dot(t_ref[k], v_ref[k],
                          preferred_element_type=jnp.float32)
    o_ref[...] = acc

  return pl.pallas_call(
      body,
      grid=(NPAD // blk,),
      in_specs=[
          pl.BlockSpec((K, blk, HID), lambda i: (0, i, 0)),
          pl.BlockSpec((K, HID, CPAD), lambda i: (0, 0, 0)),
      ],
      out_specs=pl.BlockSpec((blk, CPAD), lambda i: (i, 0)),
      out_shape=jax.ShapeDtypeStruct((NPAD, CPAD), jnp.float32),
  )(tstack, v)


def _tc_head(outpre):
  blk = 1024

  def body(x_ref, o_ref):
    x = x_ref[...]
    h = jnp.where(x > 0, x, jnp.exp(x) - 1.0)
    mask = lax.broadcasted_iota(jnp.int32, (blk, CPAD), 1) < CLS
    neg = jnp.float32(-1e30)
    hm = jnp.where(mask, h, neg)
    mx = jnp.max(hm, axis=1, keepdims=True)
    ex = jnp.where(mask, jnp.exp(h - mx), 0.0)
    lse = jnp.log(jnp.sum(ex, axis=1, keepdims=True))
    o_ref[...] = h - mx - lse

  return pl.pallas_call(
      body,
      grid=(NPAD // blk,),
      in_specs=[pl.BlockSpec((blk, CPAD), lambda i: (i, 0))],
      out_specs=pl.BlockSpec((blk, CPAD), lambda i: (i, 0)),
      out_shape=jax.ShapeDtypeStruct((NPAD, CPAD), jnp.float32),
  )(outpre)


@jax.jit
def kernel(x, edge_index, W1, theta1, W2, theta2):
  E = edge_index.shape[1]
  dummy = NPAD - 1
  src = jnp.concatenate(
      [edge_index[0], jnp.full((EPAD - E,), dummy, jnp.int32)])
  dst = jnp.concatenate(
      [edge_index[1], jnp.full((EPAD - E,), dummy, jnp.int32)])
  src2d = src.reshape(EPAD // ECHUNK, ECHUNK)
  dst2d = dst.reshape(EPAD // ECHUNK, ECHUNK)

  xpad = jnp.pad(x, ((0, NPAD - N), (0, 0)))
  v = jnp.einsum("hk,hfc->kfc", theta1, W2.reshape(HEADS, HID, CLS))
  vpad = jnp.pad(v, ((0, 0), (0, 0), (0, CPAD - CLS)))
  th_dummy = jnp.zeros((1, K), jnp.float32)
  theta2p = theta2.astype(jnp.float32)

  xw1 = _tc_matmul(xpad, W1)
  deg = _sc_degrees(src2d, dst2d)
  fg = _tc_fg(deg)
  _, tstack, _ = _recurrence(HID, False)(
      src2d, dst2d, xw1, fg, th_dummy)
  xw2 = _tc_combine(tstack, vpad)
  outpre, _, _ = _recurrence(CPAD, True)(
      src2d, dst2d, xw2, fg, theta2p)
  out = _tc_head(outpre)
  return out[:N, :CLS]


# speedup vs baseline: 10.4126x; 1.4247x over previous
import functools

import jax
import jax.numpy as jnp
from jax import lax
from jax.experimental import pallas as pl
from jax.experimental.pallas import tpu as pltpu
from jax.experimental.pallas import tpu_sc as plsc

N = 10000
D = 128
HID = 64
HEADS = 4
CLS = 7
K = 16

NPAD = 10240
EPAD = 327680
CPAD = 16
TILES = 16
NODES_PT = NPAD // TILES
ECHUNK = 128
ROWS_PT = EPAD // ECHUNK // TILES
NBLK = 128
NBLKS_PT = NODES_PT // NBLK
NB = 4


def _splat(v16, i):
  idx = jnp.full((16,), i, jnp.int32)
  return v16.at[idx].get(mode="promise_in_bounds")


def _edge_phase(src_v, dst_v, tprime_h, accum_s, buf_v, gsem, ssem):
  def _gather(c):
    s = c % NB
    return pltpu.make_async_copy(
        tprime_h.at[src_v.at[c]], buf_v.at[s], gsem.at[s])

  def _scatter(c):
    s = c % NB
    return pltpu.make_async_copy(
        buf_v.at[s], accum_s.at[dst_v.at[c]], ssem.at[s])

  _gather(0).start()
  _gather(1).start()

  def body(c, carry):
    _gather(c).wait()
    _scatter(c).start(add=True)

    @pl.when(c + 2 < ROWS_PT)
    def _():
      @pl.when(c >= 2)
      def _():
        _scatter(c - 2).wait()
      _gather(c + 2).start()
    return carry
  lax.fori_loop(0, ROWS_PT, body, 0)

  def drain(x, carry):
    _scatter(x).wait()
    return carry
  lax.fori_loop(ROWS_PT - 4, ROWS_PT, drain, 0)


def _sc_degrees(src2d, dst2d):
  def body(src_h, dst_h, deg_h, degs_s, degd_s, src_v, dst_v, ones_v, dv_v):
    cid = lax.axis_index("c")
    t = lax.axis_index("s")

    @pl.when(cid == 0)
    def _core0():
      nbase = t * NODES_PT
      pltpu.sync_copy(src_h.at[pl.ds(t * ROWS_PT, ROWS_PT)], src_v)
      pltpu.sync_copy(dst_h.at[pl.ds(t * ROWS_PT, ROWS_PT)], dst_v)

      def fill_ones(j, c):
        ones_v[pl.ds(j * 16, 16)] = jnp.full((16,), 1.0, jnp.float32)
        return c
      lax.fori_loop(0, ECHUNK // 16, fill_ones, 0)

      def fill_dv(j, c):
        dv_v[pl.ds(j * 16, 16)] = jnp.zeros((16,), jnp.float32)
        return c
      lax.fori_loop(0, NODES_PT // 16, fill_dv, 0)

      pltpu.sync_copy(dv_v, degs_s.at[pl.ds(nbase, NODES_PT)])
      pltpu.sync_copy(dv_v, degd_s.at[pl.ds(nbase, NODES_PT)])
      plsc.subcore_barrier()

      def degbody(c, carry):
        pltpu.sync_copy(ones_v, degs_s.at[src_v.at[c]], add=True)
        pltpu.sync_copy(ones_v, degd_s.at[dst_v.at[c]], add=True)
        return carry
      lax.fori_loop(0, ROWS_PT, degbody, 0)

      plsc.subcore_barrier()
      pltpu.sync_copy(degs_s.at[pl.ds(nbase, NODES_PT)],
                      deg_h.at[0, pl.ds(nbase, NODES_PT)])
      pltpu.sync_copy(degd_s.at[pl.ds(nbase, NODES_PT)],
                      deg_h.at[1, pl.ds(nbase, NODES_PT)])

  mesh = plsc.VectorSubcoreMesh(core_axis_name="c", subcore_axis_name="s")
  f = pl.kernel(
      body,
      out_type=[jax.ShapeDtypeStruct((2, NPAD), jnp.float32)],
      mesh=mesh,
      compiler_params=pltpu.CompilerParams(use_tc_tiling_on_sc=False),
      scratch_types=[
          pltpu.VMEM_SHARED((NPAD,), jnp.float32),
          pltpu.VMEM_SHARED((NPAD,), jnp.float32),
          pltpu.VMEM((ROWS_PT, ECHUNK), jnp.int32),
          pltpu.VMEM((ROWS_PT, ECHUNK), jnp.int32),
          pltpu.VMEM((ECHUNK,), jnp.float32),
          pltpu.VMEM((NODES_PT,), jnp.float32),
      ],
  )
  return f(src2d, dst2d)[0]


def _tc_fg(deg):
  def body(d_ref, o_ref):
    o_ref[...] = lax.rsqrt(jnp.maximum(d_ref[...], 1.0))

  return pl.pallas_call(
      body,
      out_shape=jax.ShapeDtypeStruct((2, NPAD), jnp.float32),
  )(deg)


def _recurrence_split(fdim2):
  def body(src_h, dst_h, x0_h, fg_h, tstack_h, tprime_h,
           accum_s, src_v, dst_v, f_v, g_v, buf_v, gsem, ssem):
    cid = lax.axis_index("c")
    t = lax.axis_index("s")
    nbase = t * NODES_PT
    pltpu.sync_copy(src_h.at[pl.ds(t * ROWS_PT, ROWS_PT)], src_v)
    pltpu.sync_copy(dst_h.at[pl.ds(t * ROWS_PT, ROWS_PT)], dst_v)
    pltpu.sync_copy(fg_h.at[0, pl.ds(nbase, NODES_PT)], f_v)
    pltpu.sync_copy(fg_h.at[1, pl.ds(nbase, NODES_PT)], g_v)
    my_x0 = x0_h.at[cid]
    my_ts = tstack_h.at[cid]
    my_tp = tprime_h.at[cid]

    def _sl(b):
      return pl.ds(nbase + b * NBLK, NBLK)

    def fill_zero(r, c):
      for fb in range(fdim2 // 16):
        buf_v[2, r, pl.ds(fb * 16, 16)] = jnp.zeros((16,), jnp.float32)
      return c
    lax.fori_loop(0, NBLK, fill_zero, 0)

    def zacc(b, c):
      pltpu.sync_copy(buf_v.at[2], accum_s.at[_sl(b)])
      return c
    lax.fori_loop(0, NBLKS_PT, zacc, 0)

    def initb(b, c):
      sl = _sl(b)
      pltpu.sync_copy(my_x0.at[sl], buf_v.at[0])
      pltpu.sync_copy(buf_v.at[0], my_ts.at[0, sl])

      def grp(jj, c2):
        f16 = f_v[pl.ds(b * NBLK + jj * 16, 16)]

        def lane(i2, c3):
          fs = _splat(f16, i2)
          r = jj * 16 + i2
          for fb in range(fdim2 // 16):
            row = buf_v[0, r, pl.ds(fb * 16, 16)]
            buf_v[1, r, pl.ds(fb * 16, 16)] = fs * row
          return c3
        lax.fori_loop(0, 16, lane, 0)
        return c2
      lax.fori_loop(0, NBLK // 16, grp, 0)
      pltpu.sync_copy(buf_v.at[1], my_tp.at[sl])
      return c
    lax.fori_loop(0, NBLKS_PT, initb, 0)

    plsc.subcore_barrier()

    def kstep(k, carry):
      _edge_phase(src_v, dst_v, my_tp, accum_s, buf_v, gsem, ssem)
      plsc.subcore_barrier()

      kidx = jnp.maximum(k - 2, 0)
      sp = jnp.where(k >= 2, 1.0, 0.0).astype(jnp.float32)
      ca = jnp.where(k >= 2, -2.0, -1.0).astype(jnp.float32)
      sp16 = jnp.full((16,), 1.0, jnp.float32) * sp

      lax.fori_loop(0, NBLK, fill_zero, 0)

      def nodeb(b, c):
        sl = _sl(b)
        pltpu.sync_copy(accum_s.at[sl], buf_v.at[0])
        pltpu.sync_copy(buf_v.at[2], accum_s.at[sl])
        pltpu.sync_copy(my_ts.at[kidx, sl], buf_v.at[1])

        def grp(jj, c2):
          g16 = g_v[pl.ds(b * NBLK + jj * 16, 16)] * ca
          f16 = f_v[pl.ds(b * NBLK + jj * 16, 16)]

          def lane(i2, c3):
            cg = _splat(g16, i2)
            fs = _splat(f16, i2)
            r = jj * 16 + i2
            for fb in range(fdim2 // 16):
              a = buf_v[0, r, pl.ds(fb * 16, 16)]
              tp = buf_v[1, r, pl.ds(fb * 16, 16)]
              tn = cg * a - sp16 * tp
              buf_v[1, r, pl.ds(fb * 16, 16)] = tn
              buf_v[0, r, pl.ds(fb * 16, 16)] = fs * tn
            return c3
          lax.fori_loop(0, 16, lane, 0)
          return c2
        lax.fori_loop(0, NBLK // 16, grp, 0)

        pltpu.sync_copy(buf_v.at[1], my_ts.at[k, sl])
        pltpu.sync_copy(buf_v.at[0], my_tp.at[sl])
        return c
      lax.fori_loop(0, NBLKS_PT, nodeb, 0)
      plsc.subcore_barrier()
      return carry

    lax.fori_loop(1, K, kstep, 0)

  mesh = plsc.VectorSubcoreMesh(core_axis_name="c", subcore_axis_name="s")
  return pl.kernel(
      body,
      out_type=[
          jax.ShapeDtypeStruct((2, K, NPAD, fdim2), jnp.float32),
          jax.ShapeDtypeStruct((2, NPAD, fdim2), jnp.float32),
      ],
      mesh=mesh,
      compiler_params=pltpu.CompilerParams(use_tc_tiling_on_sc=False),
      scratch_types=[
          pltpu.VMEM_SHARED((NPAD, fdim2), jnp.float32),
          pltpu.VMEM((ROWS_PT, ECHUNK), jnp.int32),
          pltpu.VMEM((ROWS_PT, ECHUNK), jnp.int32),
          pltpu.VMEM((NODES_PT,), jnp.float32),
          pltpu.VMEM((NODES_PT,), jnp.float32),
          pltpu.VMEM((NB, ECHUNK, fdim2), jnp.float32),
          pltpu.SemaphoreType.DMA((NB,)),
          pltpu.SemaphoreType.DMA((NB,)),
      ],
  )


def _recurrence(fdim, with_out):
  def body(src_h, dst_h, x0_h, fg_h, th_h, out_h, tstack_h, tprime_h,
           accum_s, src_v, dst_v, f_v, g_v, th_v, buf_v, gsem, ssem):
    cid = lax.axis_index("c")
    t = lax.axis_index("s")

    @pl.when(cid == 0)
    def _core0():
      nbase = t * NODES_PT
      pltpu.sync_copy(src_h.at[pl.ds(t * ROWS_PT, ROWS_PT)], src_v)
      pltpu.sync_copy(dst_h.at[pl.ds(t * ROWS_PT, ROWS_PT)], dst_v)
      pltpu.sync_copy(fg_h.at[0, pl.ds(nbase, NODES_PT)], f_v)
      pltpu.sync_copy(fg_h.at[1, pl.ds(nbase, NODES_PT)], g_v)
      pltpu.sync_copy(th_h.at[0], th_v)
      th16 = th_v[...]
      th0 = _splat(th16, 0)

      def _sl(b):
        return pl.ds(nbase + b * NBLK, NBLK)

      def fill_zero(r, c):
        for fb in range(fdim // 16):
          buf_v[2, r, pl.ds(fb * 16, 16)] = jnp.zeros((16,), jnp.float32)
        return c
      lax.fori_loop(0, NBLK, fill_zero, 0)

      def zacc(b, c):
        pltpu.sync_copy(buf_v.at[2], accum_s.at[_sl(b)])
        return c
      lax.fori_loop(0, NBLKS_PT, zacc, 0)

      def initb(b, c):
        sl = _sl(b)
        pltpu.sync_copy(x0_h.at[sl], buf_v.at[0])
        pltpu.sync_copy(buf_v.at[0], tstack_h.at[0, sl])

        def grp(jj, c2):
          f16 = f_v[pl.ds(b * NBLK + jj * 16, 16)]

          def lane(i2, c3):
            fs = _splat(f16, i2)
            r = jj * 16 + i2
            for fb in range(fdim // 16):
              row = buf_v[0, r, pl.ds(fb * 16, 16)]
              buf_v[1, r, pl.ds(fb * 16, 16)] = fs * row
              if with_out:
                buf_v[3, r, pl.ds(fb * 16, 16)] = th0 * row
            return c3
          lax.fori_loop(0, 16, lane, 0)
          return c2
        lax.fori_loop(0, NBLK // 16, grp, 0)
        pltpu.sync_copy(buf_v.at[1], tprime_h.at[sl])
        if with_out:
          pltpu.sync_copy(buf_v.at[3], out_h.at[sl])
        return c
      lax.fori_loop(0, NBLKS_PT, initb, 0)

      plsc.subcore_barrier()

      def kstep(k, carry):
        _edge_phase(src_v, dst_v, tprime_h, accum_s, buf_v, gsem, ssem)
        plsc.subcore_barrier()

        kidx = jnp.maximum(k - 2, 0)
        sp = jnp.where(k >= 2, 1.0, 0.0).astype(jnp.float32)
        ca = jnp.where(k >= 2, -2.0, -1.0).astype(jnp.float32)
        sp16 = jnp.full((16,), 1.0, jnp.float32) * sp
        thk = _splat(th16, k)

        lax.fori_loop(0, NBLK, fill_zero, 0)

        def nodeb(b, c):
          sl = _sl(b)
          pltpu.sync_copy(accum_s.at[sl], buf_v.at[0])
          pltpu.sync_copy(buf_v.at[2], accum_s.at[sl])
          pltpu.sync_copy(tstack_h.at[kidx, sl], buf_v.at[1])
          if with_out:
            pltpu.sync_copy(out_h.at[sl], buf_v.at[3])

          def grp(jj, c2):
            g16 = g_v[pl.ds(b * NBLK + jj * 16, 16)] * ca
            f16 = f_v[pl.ds(b * NBLK + jj * 16, 16)]

            def lane(i2, c3):
              cg = _splat(g16, i2)
              fs = _splat(f16, i2)
              r = jj * 16 + i2
              for fb in range(fdim // 16):
                a = buf_v[0, r, pl.ds(fb * 16, 16)]
                tp = buf_v[1, r, pl.ds(fb * 16, 16)]
                tn = cg * a - sp16 * tp
                buf_v[1, r, pl.ds(fb * 16, 16)] = tn
                buf_v[0, r, pl.ds(fb * 16, 16)] = fs * tn
                if with_out:
                  buf_v[3, r, pl.ds(fb * 16, 16)] = (
                      buf_v[3, r, pl.ds(fb * 16, 16)] + thk * tn)
              return c3
            lax.fori_loop(0, 16, lane, 0)
            return c2
          lax.fori_loop(0, NBLK // 16, grp, 0)

          pltpu.sync_copy(buf_v.at[1], tstack_h.at[k, sl])
          pltpu.sync_copy(buf_v.at[0], tprime_h.at[sl])
          if with_out:
            pltpu.sync_copy(buf_v.at[3], out_h.at[sl])
          return c
        lax.fori_loop(0, NBLKS_PT, nodeb, 0)
        plsc.subcore_barrier()
        return carry

      lax.fori_loop(1, K, kstep, 0)

  mesh = plsc.VectorSubcoreMesh(core_axis_name="c", subcore_axis_name="s")
  return pl.kernel(
      body,
      out_type=[
          jax.ShapeDtypeStruct((NPAD, fdim), jnp.float32),
          jax.ShapeDtypeStruct((K, NPAD, fdim), jnp.float32),
          jax.ShapeDtypeStruct((NPAD, fdim), jnp.float32),
      ],
      mesh=mesh,
      compiler_params=pltpu.CompilerParams(use_tc_tiling_on_sc=False),
      scratch_types=[
          pltpu.VMEM_SHARED((NPAD, fdim), jnp.float32),
          pltpu.VMEM((ROWS_PT, ECHUNK), jnp.int32),
          pltpu.VMEM((ROWS_PT, ECHUNK), jnp.int32),
          pltpu.VMEM((NODES_PT,), jnp.float32),
          pltpu.VMEM((NODES_PT,), jnp.float32),
          pltpu.VMEM((16,), jnp.float32),
          pltpu.VMEM((NB, ECHUNK, fdim), jnp.float32),
          pltpu.SemaphoreType.DMA((NB,)),
          pltpu.SemaphoreType.DMA((NB,)),
      ],
  )


def _tc_matmul(x, w):
  blk = 1024

  def body(x_ref, w_ref, o_ref):
    o_ref[...] = jnp.dot(x_ref[...], w_ref[...],
                         preferred_element_type=jnp.float32)

  return pl.pallas_call(
      body,
      grid=(NPAD // blk,),
      in_specs=[
          pl.BlockSpec((blk, D), lambda i: (i, 0)),
          pl.BlockSpec((D, HID), lambda i: (0, 0)),
      ],
      out_specs=pl.BlockSpec((blk, HID), lambda i: (i, 0)),
      out_shape=jax.ShapeDtypeStruct((NPAD, HID), jnp.float32),
  )(x, w)


def _tc_combine(tstack, v):
  blk = 1024
  f2 = HID // 2

  def body(t_ref, v_ref, o_ref):
    acc = jnp.zeros((blk, CPAD), jnp.float32)
    for h in range(2):
      for k in range(K):
        acc = acc + jnp.dot(t_ref[h, k], v_ref[h, k],
                            preferred_element_type=jnp.float32)
    o_ref[...] = acc

  return pl.pallas_call(
      body,
      grid=(NPAD // blk,),
      in_specs=[
          pl.BlockSpec((2, K, blk, f2), lambda i: (0, 0, i, 0)),
          pl.BlockSpec((2, K, f2, CPAD), lambda i: (0, 0, 0, 0)),
      ],
      out_specs=pl.BlockSpec((blk, CPAD), lambda i: (i, 0)),
      out_shape=jax.ShapeDtypeStruct((NPAD, CPAD), jnp.float32),
  )(tstack, v)


def _tc_head(outpre):
  blk = 1024

  def body(x_ref, o_ref):
    x = x_ref[...]
    h = jnp.where(x > 0, x, jnp.exp(x) - 1.0)
    mask = lax.broadcasted_iota(jnp.int32, (blk, CPAD), 1) < CLS
    neg = jnp.float32(-1e30)
    hm = jnp.where(mask, h, neg)
    mx = jnp.max(hm, axis=1, keepdims=True)
    ex = jnp.where(mask, jnp.exp(h - mx), 0.0)
    lse = jnp.log(jnp.sum(ex, axis=1, keepdims=True))
    o_ref[...] = h - mx - lse

  return pl.pallas_call(
      body,
      grid=(NPAD // blk,),
      in_specs=[pl.BlockSpec((blk, CPAD), lambda i: (i, 0))],
      out_specs=pl.BlockSpec((blk, CPAD), lambda i: (i, 0)),
      out_shape=jax.ShapeDtypeStruct((NPAD, CPAD), jnp.float32),
  )(outpre)


@jax.jit
def kernel(x, edge_index, W1, theta1, W2, theta2):
  E = edge_index.shape[1]
  dummy = NPAD - 1
  src = jnp.concatenate(
      [edge_index[0], jnp.full((EPAD - E,), dummy, jnp.int32)])
  dst = jnp.concatenate(
      [edge_index[1], jnp.full((EPAD - E,), dummy, jnp.int32)])
  src2d = src.reshape(EPAD // ECHUNK, ECHUNK)
  dst2d = dst.reshape(EPAD // ECHUNK, ECHUNK)

  xpad = jnp.pad(x, ((0, NPAD - N), (0, 0)))
  v = jnp.einsum("hk,hfc->kfc", theta1, W2.reshape(HEADS, HID, CLS))
  vpad = jnp.pad(v, ((0, 0), (0, 0), (0, CPAD - CLS)))
  th_dummy = jnp.zeros((1, K), jnp.float32)
  theta2p = theta2.astype(jnp.float32)

  xw1 = _tc_matmul(xpad, W1)
  deg = _sc_degrees(src2d, dst2d)
  fg = _tc_fg(deg)
  f2 = HID // 2
  xw1s = jnp.stack([xw1[:, :f2], xw1[:, f2:]])
  vs = jnp.stack([vpad[:, :f2, :], vpad[:, f2:, :]])
  tstack, _ = _recurrence_split(f2)(src2d, dst2d, xw1s, fg)
  xw2 = _tc_combine(tstack, vs)
  outpre, _, _ = _recurrence(CPAD, True)(
      src2d, dst2d, xw2, fg, theta2p)
  out = _tc_head(outpre)
  return out[:N, :CLS]


# speedup vs baseline: 11.2034x; 1.0759x over previous
import functools

import jax
import jax.numpy as jnp
from jax import lax
from jax.experimental import pallas as pl
from jax.experimental.pallas import tpu as pltpu
from jax.experimental.pallas import tpu_sc as plsc

N = 10000
D = 128
HID = 64
HEADS = 4
CLS = 7
K = 16

NPAD = 10240
EPAD = 327680
CPAD = 16
TILES = 16
NODES_PT = NPAD // TILES
ECHUNK = 128
ROWS_PT = EPAD // ECHUNK // TILES
NBLK = 128
NBLKS_PT = NODES_PT // NBLK
NB = 6
NG = NB // 2


def _splat(v16, i):
  idx = jnp.full((16,), i, jnp.int32)
  return v16.at[idx].get(mode="promise_in_bounds")


def _edge_phase(src_v, dst_v, tprime_h, accum_s, buf_v, gsem, ssem):
  def _gather(c):
    s = c % NB
    return pltpu.make_async_copy(
        tprime_h.at[src_v.at[c]], buf_v.at[s], gsem.at[s])

  def _scatter(c):
    s = c % NB
    return pltpu.make_async_copy(
        buf_v.at[s], accum_s.at[dst_v.at[c]], ssem.at[s])

  def prolog(j, carry):
    _gather(j).start()
    return carry
  lax.fori_loop(0, NG, prolog, 0)

  def body(c, carry):
    _gather(c).wait()
    _scatter(c).start(add=True)

    @pl.when(c + NG < ROWS_PT)
    def _():
      @pl.when(c >= NB - NG)
      def _():
        _scatter(c - (NB - NG)).wait()
      _gather(c + NG).start()
    return carry
  lax.fori_loop(0, ROWS_PT, body, 0)

  def drain(x, carry):
    _scatter(x).wait()
    return carry
  lax.fori_loop(ROWS_PT - NB, ROWS_PT, drain, 0)


def _sc_degrees(src2d, dst2d):
  def body(src_h, dst_h, deg_h, degs_s, degd_s, src_v, dst_v, ones_v, dv_v):
    cid = lax.axis_index("c")
    t = lax.axis_index("s")

    @pl.when(cid == 0)
    def _core0():
      nbase = t * NODES_PT
      pltpu.sync_copy(src_h.at[pl.ds(t * ROWS_PT, ROWS_PT)], src_v)
      pltpu.sync_copy(dst_h.at[pl.ds(t * ROWS_PT, ROWS_PT)], dst_v)

      def fill_ones(j, c):
        ones_v[pl.ds(j * 16, 16)] = jnp.full((16,), 1.0, jnp.float32)
        return c
      lax.fori_loop(0, ECHUNK // 16, fill_ones, 0)

      def fill_dv(j, c):
        dv_v[pl.ds(j * 16, 16)] = jnp.zeros((16,), jnp.float32)
        return c
      lax.fori_loop(0, NODES_PT // 16, fill_dv, 0)

      pltpu.sync_copy(dv_v, degs_s.at[pl.ds(nbase, NODES_PT)])
      pltpu.sync_copy(dv_v, degd_s.at[pl.ds(nbase, NODES_PT)])
      plsc.subcore_barrier()

      def degbody(c, carry):
        pltpu.sync_copy(ones_v, degs_s.at[src_v.at[c]], add=True)
        pltpu.sync_copy(ones_v, degd_s.at[dst_v.at[c]], add=True)
        return carry
      lax.fori_loop(0, ROWS_PT, degbody, 0)

      plsc.subcore_barrier()
      pltpu.sync_copy(degs_s.at[pl.ds(nbase, NODES_PT)],
                      deg_h.at[0, pl.ds(nbase, NODES_PT)])
      pltpu.sync_copy(degd_s.at[pl.ds(nbase, NODES_PT)],
                      deg_h.at[1, pl.ds(nbase, NODES_PT)])

  mesh = plsc.VectorSubcoreMesh(core_axis_name="c", subcore_axis_name="s")
  f = pl.kernel(
      body,
      out_type=[jax.ShapeDtypeStruct((2, NPAD), jnp.float32)],
      mesh=mesh,
      compiler_params=pltpu.CompilerParams(use_tc_tiling_on_sc=False),
      scratch_types=[
          pltpu.VMEM_SHARED((NPAD,), jnp.float32),
          pltpu.VMEM_SHARED((NPAD,), jnp.float32),
          pltpu.VMEM((ROWS_PT, ECHUNK), jnp.int32),
          pltpu.VMEM((ROWS_PT, ECHUNK), jnp.int32),
          pltpu.VMEM((ECHUNK,), jnp.float32),
          pltpu.VMEM((NODES_PT,), jnp.float32),
      ],
  )
  return f(src2d, dst2d)[0]


def _tc_fg(deg):
  def body(d_ref, o_ref):
    o_ref[...] = lax.rsqrt(jnp.maximum(d_ref[...], 1.0))

  return pl.pallas_call(
      body,
      out_shape=jax.ShapeDtypeStruct((2, NPAD), jnp.float32),
  )(deg)


def _recurrence_split(fdim2):
  def body(src_h, dst_h, x0_h, fg_h, tstack_h, tprime_h,
           accum_s, src_v, dst_v, f_v, g_v, buf_v, gsem, ssem):
    cid = lax.axis_index("c")
    t = lax.axis_index("s")
    nbase = t * NODES_PT
    pltpu.sync_copy(src_h.at[pl.ds(t * ROWS_PT, ROWS_PT)], src_v)
    pltpu.sync_copy(dst_h.at[pl.ds(t * ROWS_PT, ROWS_PT)], dst_v)
    pltpu.sync_copy(fg_h.at[0, pl.ds(nbase, NODES_PT)], f_v)
    pltpu.sync_copy(fg_h.at[1, pl.ds(nbase, NODES_PT)], g_v)
    my_x0 = x0_h.at[cid]
    my_ts = tstack_h.at[cid]
    my_tp = tprime_h.at[cid]

    def _sl(b):
      return pl.ds(nbase + b * NBLK, NBLK)

    def fill_zero(r, c):
      for fb in range(fdim2 // 16):
        buf_v[2, r, pl.ds(fb * 16, 16)] = jnp.zeros((16,), jnp.float32)
      return c
    lax.fori_loop(0, NBLK, fill_zero, 0)

    def zacc(b, c):
      pltpu.sync_copy(buf_v.at[2], accum_s.at[_sl(b)])
      return c
    lax.fori_loop(0, NBLKS_PT, zacc, 0)

    def initb(b, c):
      sl = _sl(b)
      pltpu.sync_copy(my_x0.at[sl], buf_v.at[0])
      pltpu.sync_copy(buf_v.at[0], my_ts.at[0, sl])

      def grp(jj, c2):
        f16 = f_v[pl.ds(b * NBLK + jj * 16, 16)]

        def lane(i2, c3):
          fs = _splat(f16, i2)
          r = jj * 16 + i2
          for fb in range(fdim2 // 16):
            row = buf_v[0, r, pl.ds(fb * 16, 16)]
            buf_v[1, r, pl.ds(fb * 16, 16)] = fs * row
          return c3
        lax.fori_loop(0, 16, lane, 0)
        return c2
      lax.fori_loop(0, NBLK // 16, grp, 0)
      pltpu.sync_copy(buf_v.at[1], my_tp.at[sl])
      return c
    lax.fori_loop(0, NBLKS_PT, initb, 0)

    plsc.subcore_barrier()

    def kstep(k, carry):
      _edge_phase(src_v, dst_v, my_tp, accum_s, buf_v, gsem, ssem)
      plsc.subcore_barrier()

      kidx = jnp.maximum(k - 2, 0)
      sp = jnp.where(k >= 2, 1.0, 0.0).astype(jnp.float32)
      ca = jnp.where(k >= 2, -2.0, -1.0).astype(jnp.float32)
      sp16 = jnp.full((16,), 1.0, jnp.float32) * sp

      lax.fori_loop(0, NBLK, fill_zero, 0)

      def nodeb(b, c):
        sl = _sl(b)
        pltpu.sync_copy(accum_s.at[sl], buf_v.at[0])
        pltpu.sync_copy(buf_v.at[2], accum_s.at[sl])
        pltpu.sync_copy(my_ts.at[kidx, sl], buf_v.at[1])

        def grp(jj, c2):
          g16 = g_v[pl.ds(b * NBLK + jj * 16, 16)] * ca
          f16 = f_v[pl.ds(b * NBLK + jj * 16, 16)]

          def lane(i2, c3):
            cg = _splat(g16, i2)
            fs = _splat(f16, i2)
            r = jj * 16 + i2
            for fb in range(fdim2 // 16):
              a = buf_v[0, r, pl.ds(fb * 16, 16)]
              tp = buf_v[1, r, pl.ds(fb * 16, 16)]
              tn = cg * a - sp16 * tp
              buf_v[1, r, pl.ds(fb * 16, 16)] = tn
              buf_v[0, r, pl.ds(fb * 16, 16)] = fs * tn
            return c3
          lax.fori_loop(0, 16, lane, 0)
          return c2
        lax.fori_loop(0, NBLK // 16, grp, 0)

        pltpu.sync_copy(buf_v.at[1], my_ts.at[k, sl])
        pltpu.sync_copy(buf_v.at[0], my_tp.at[sl])
        return c
      lax.fori_loop(0, NBLKS_PT, nodeb, 0)
      plsc.subcore_barrier()
      return carry

    lax.fori_loop(1, K, kstep, 0)

  mesh = plsc.VectorSubcoreMesh(core_axis_name="c", subcore_axis_name="s")
  return pl.kernel(
      body,
      out_type=[
          jax.ShapeDtypeStruct((2, K, NPAD, fdim2), jnp.float32),
          jax.ShapeDtypeStruct((2, NPAD, fdim2), jnp.float32),
      ],
      mesh=mesh,
      compiler_params=pltpu.CompilerParams(use_tc_tiling_on_sc=False),
      scratch_types=[
          pltpu.VMEM_SHARED((NPAD, fdim2), jnp.float32),
          pltpu.VMEM((ROWS_PT, ECHUNK), jnp.int32),
          pltpu.VMEM((ROWS_PT, ECHUNK), jnp.int32),
          pltpu.VMEM((NODES_PT,), jnp.float32),
          pltpu.VMEM((NODES_PT,), jnp.float32),
          pltpu.VMEM((NB, ECHUNK, fdim2), jnp.float32),
          pltpu.SemaphoreType.DMA((NB,)),
          pltpu.SemaphoreType.DMA((NB,)),
      ],
  )


def _recurrence(fdim, with_out):
  def body(src_h, dst_h, x0_h, fg_h, th_h, out_h, tstack_h, tprime_h,
           accum_s, src_v, dst_v, f_v, g_v, th_v, buf_v, gsem, ssem):
    cid = lax.axis_index("c")
    t = lax.axis_index("s")

    @pl.when(cid == 0)
    def _core0():
      nbase = t * NODES_PT
      pltpu.sync_copy(src_h.at[pl.ds(t * ROWS_PT, ROWS_PT)], src_v)
      pltpu.sync_copy(dst_h.at[pl.ds(t * ROWS_PT, ROWS_PT)], dst_v)
      pltpu.sync_copy(fg_h.at[0, pl.ds(nbase, NODES_PT)], f_v)
      pltpu.sync_copy(fg_h.at[1, pl.ds(nbase, NODES_PT)], g_v)
      pltpu.sync_copy(th_h.at[0], th_v)
      th16 = th_v[...]
      th0 = _splat(th16, 0)

      def _sl(b):
        return pl.ds(nbase + b * NBLK, NBLK)

      def fill_zero(r, c):
        for fb in range(fdim // 16):
          buf_v[2, r, pl.ds(fb * 16, 16)] = jnp.zeros((16,), jnp.float32)
        return c
      lax.fori_loop(0, NBLK, fill_zero, 0)

      def zacc(b, c):
        pltpu.sync_copy(buf_v.at[2], accum_s.at[_sl(b)])
        return c
      lax.fori_loop(0, NBLKS_PT, zacc, 0)

      def initb(b, c):
        sl = _sl(b)
        pltpu.sync_copy(x0_h.at[sl], buf_v.at[0])
        pltpu.sync_copy(buf_v.at[0], tstack_h.at[0, sl])

        def grp(jj, c2):
          f16 = f_v[pl.ds(b * NBLK + jj * 16, 16)]

          def lane(i2, c3):
            fs = _splat(f16, i2)
            r = jj * 16 + i2
            for fb in range(fdim // 16):
              row = buf_v[0, r, pl.ds(fb * 16, 16)]
              buf_v[1, r, pl.ds(fb * 16, 16)] = fs * row
              if with_out:
                buf_v[3, r, pl.ds(fb * 16, 16)] = th0 * row
            return c3
          lax.fori_loop(0, 16, lane, 0)
          return c2
        lax.fori_loop(0, NBLK // 16, grp, 0)
        pltpu.sync_copy(buf_v.at[1], tprime_h.at[sl])
        if with_out:
          pltpu.sync_copy(buf_v.at[3], out_h.at[sl])
        return c
      lax.fori_loop(0, NBLKS_PT, initb, 0)

      plsc.subcore_barrier()

      def kstep(k, carry):
        _edge_phase(src_v, dst_v, tprime_h, accum_s, buf_v, gsem, ssem)
        plsc.subcore_barrier()

        kidx = jnp.maximum(k - 2, 0)
        sp = jnp.where(k >= 2, 1.0, 0.0).astype(jnp.float32)
        ca = jnp.where(k >= 2, -2.0, -1.0).astype(jnp.float32)
        sp16 = jnp.full((16,), 1.0, jnp.float32) * sp
        thk = _splat(th16, k)

        lax.fori_loop(0, NBLK, fill_zero, 0)

        def nodeb(b, c):
          sl = _sl(b)
          pltpu.sync_copy(accum_s.at[sl], buf_v.at[0])
          pltpu.sync_copy(buf_v.at[2], accum_s.at[sl])
          pltpu.sync_copy(tstack_h.at[kidx, sl], buf_v.at[1])
          if with_out:
            pltpu.sync_copy(out_h.at[sl], buf_v.at[3])

          def grp(jj, c2):
            g16 = g_v[pl.ds(b * NBLK + jj * 16, 16)] * ca
            f16 = f_v[pl.ds(b * NBLK + jj * 16, 16)]

            def lane(i2, c3):
              cg = _splat(g16, i2)
              fs = _splat(f16, i2)
              r = jj * 16 + i2
              for fb in range(fdim // 16):
                a = buf_v[0, r, pl.ds(fb * 16, 16)]
                tp = buf_v[1, r, pl.ds(fb * 16, 16)]
                tn = cg * a - sp16 * tp
                buf_v[1, r, pl.ds(fb * 16, 16)] = tn
                buf_v[0, r, pl.ds(fb * 16, 16)] = fs * tn
                if with_out:
                  buf_v[3, r, pl.ds(fb * 16, 16)] = (
                      buf_v[3, r, pl.ds(fb * 16, 16)] + thk * tn)
              return c3
            lax.fori_loop(0, 16, lane, 0)
            return c2
          lax.fori_loop(0, NBLK // 16, grp, 0)

          pltpu.sync_copy(buf_v.at[1], tstack_h.at[k, sl])
          pltpu.sync_copy(buf_v.at[0], tprime_h.at[sl])
          if with_out:
            pltpu.sync_copy(buf_v.at[3], out_h.at[sl])
          return c
        lax.fori_loop(0, NBLKS_PT, nodeb, 0)
        plsc.subcore_barrier()
        return carry

      lax.fori_loop(1, K, kstep, 0)

  mesh = plsc.VectorSubcoreMesh(core_axis_name="c", subcore_axis_name="s")
  return pl.kernel(
      body,
      out_type=[
          jax.ShapeDtypeStruct((NPAD, fdim), jnp.float32),
          jax.ShapeDtypeStruct((K, NPAD, fdim), jnp.float32),
          jax.ShapeDtypeStruct((NPAD, fdim), jnp.float32),
      ],
      mesh=mesh,
      compiler_params=pltpu.CompilerParams(use_tc_tiling_on_sc=False),
      scratch_types=[
          pltpu.VMEM_SHARED((NPAD, fdim), jnp.float32),
          pltpu.VMEM((ROWS_PT, ECHUNK), jnp.int32),
          pltpu.VMEM((ROWS_PT, ECHUNK), jnp.int32),
          pltpu.VMEM((NODES_PT,), jnp.float32),
          pltpu.VMEM((NODES_PT,), jnp.float32),
          pltpu.VMEM((16,), jnp.float32),
          pltpu.VMEM((NB, ECHUNK, fdim), jnp.float32),
          pltpu.SemaphoreType.DMA((NB,)),
          pltpu.SemaphoreType.DMA((NB,)),
      ],
  )


def _tc_matmul(x, w):
  blk = 1024

  def body(x_ref, w_ref, o_ref):
    o_ref[...] = jnp.dot(x_ref[...], w_ref[...],
                         preferred_element_type=jnp.float32)

  return pl.pallas_call(
      body,
      grid=(NPAD // blk,),
      in_specs=[
          pl.BlockSpec((blk, D), lambda i: (i, 0)),
          pl.BlockSpec((D, HID), lambda i: (0, 0)),
      ],
      out_specs=pl.BlockSpec((blk, HID), lambda i: (i, 0)),
      out_shape=jax.ShapeDtypeStruct((NPAD, HID), jnp.float32),
  )(x, w)


def _tc_combine(tstack, v):
  blk = 1024
  f2 = HID // 2

  def body(t_ref, v_ref, o_ref):
    acc = jnp.zeros((blk, CPAD), jnp.float32)
    for h in range(2):
      for k in range(K):
        acc = acc + jnp.dot(t_ref[h, k], v_ref[h, k],
                            preferred_element_type=jnp.float32)
    o_ref[...] = acc

  return pl.pallas_call(
      body,
      grid=(NPAD // blk,),
      in_specs=[
          pl.BlockSpec((2, K, blk, f2), lambda i: (0, 0, i, 0)),
          pl.BlockSpec((2, K, f2, CPAD), lambda i: (0, 0, 0, 0)),
      ],
      out_specs=pl.BlockSpec((blk, CPAD), lambda i: (i, 0)),
      out_shape=jax.ShapeDtypeStruct((NPAD, CPAD), jnp.float32),
  )(tstack, v)


def _tc_head(outpre):
  blk = 1024

  def body(x_ref, o_ref):
    x = x_ref[...]
    h = jnp.where(x > 0, x, jnp.exp(x) - 1.0)
    mask = lax.broadcasted_iota(jnp.int32, (blk, CPAD), 1) < CLS
    neg = jnp.float32(-1e30)
    hm = jnp.where(mask, h, neg)
    mx = jnp.max(hm, axis=1, keepdims=True)
    ex = jnp.where(mask, jnp.exp(h - mx), 0.0)
    lse = jnp.log(jnp.sum(ex, axis=1, keepdims=True))
    o_ref[...] = h - mx - lse

  return pl.pallas_call(
      body,
      grid=(NPAD // blk,),
      in_specs=[pl.BlockSpec((blk, CPAD), lambda i: (i, 0))],
      out_specs=pl.BlockSpec((blk, CPAD), lambda i: (i, 0)),
      out_shape=jax.ShapeDtypeStruct((NPAD, CPAD), jnp.float32),
  )(outpre)


@jax.jit
def kernel(x, edge_index, W1, theta1, W2, theta2):
  E = edge_index.shape[1]
  dummy = NPAD - 1
  src = jnp.concatenate(
      [edge_index[0], jnp.full((EPAD - E,), dummy, jnp.int32)])
  dst = jnp.concatenate(
      [edge_index[1], jnp.full((EPAD - E,), dummy, jnp.int32)])
  src2d = src.reshape(EPAD // ECHUNK, ECHUNK)
  dst2d = dst.reshape(EPAD // ECHUNK, ECHUNK)

  xpad = jnp.pad(x, ((0, NPAD - N), (0, 0)))
  v = jnp.einsum("hk,hfc->kfc", theta1, W2.reshape(HEADS, HID, CLS))
  vpad = jnp.pad(v, ((0, 0), (0, 0), (0, CPAD - CLS)))
  th_dummy = jnp.zeros((1, K), jnp.float32)
  theta2p = theta2.astype(jnp.float32)

  xw1 = _tc_matmul(xpad, W1)
  deg = _sc_degrees(src2d, dst2d)
  fg = _tc_fg(deg)
  f2 = HID // 2
  xw1s = jnp.stack([xw1[:, :f2], xw1[:, f2:]])
  vs = jnp.stack([vpad[:, :f2, :], vpad[:, f2:, :]])
  tstack, _ = _recurrence_split(f2)(src2d, dst2d, xw1s, fg)
  xw2 = _tc_combine(tstack, vs)
  outpre, _, _ = _recurrence(CPAD, True)(
      src2d, dst2d, xw2, fg, theta2p)
  out = _tc_head(outpre)
  return out[:N, :CLS]


# speedup vs baseline: 11.5989x; 1.0353x over previous
import functools

import jax
import jax.numpy as jnp
from jax import lax
from jax.experimental import pallas as pl
from jax.experimental.pallas import tpu as pltpu
from jax.experimental.pallas import tpu_sc as plsc

N = 10000
D = 128
HID = 64
HEADS = 4
CLS = 7
K = 16

NPAD = 10240
EPAD = 327680
CPAD = 16
TILES = 16
NODES_PT = NPAD // TILES
ECHUNK = 128
ROWS_PT = EPAD // ECHUNK // TILES
NBLK = 128
NBLKS_PT = NODES_PT // NBLK
NB = 8
NG = NB // 2


def _splat(v16, i):
  idx = jnp.full((16,), i, jnp.int32)
  return v16.at[idx].get(mode="promise_in_bounds")


def _edge_phase(src_v, dst_v, tprime_h, accum_s, buf_v, gsem, ssem):
  def _gather(c):
    s = c % NB
    return pltpu.make_async_copy(
        tprime_h.at[src_v.at[c]], buf_v.at[s], gsem.at[s])

  def _scatter(c):
    s = c % NB
    return pltpu.make_async_copy(
        buf_v.at[s], accum_s.at[dst_v.at[c]], ssem.at[s])

  def prolog(j, carry):
    _gather(j).start()
    return carry
  lax.fori_loop(0, NG, prolog, 0)

  def body(c, carry):
    _gather(c).wait()
    _scatter(c).start(add=True)

    @pl.when(c + NG < ROWS_PT)
    def _():
      @pl.when(c >= NB - NG)
      def _():
        _scatter(c - (NB - NG)).wait()
      _gather(c + NG).start()
    return carry
  lax.fori_loop(0, ROWS_PT, body, 0)

  def drain(x, carry):
    _scatter(x).wait()
    return carry
  lax.fori_loop(ROWS_PT - NB, ROWS_PT, drain, 0)


def _sc_degrees(src2d, dst2d):
  def body(src_h, dst_h, deg_h, degs_s, degd_s, src_v, dst_v, ones_v, dv_v):
    cid = lax.axis_index("c")
    t = lax.axis_index("s")

    @pl.when(cid == 0)
    def _core0():
      nbase = t * NODES_PT
      pltpu.sync_copy(src_h.at[pl.ds(t * ROWS_PT, ROWS_PT)], src_v)
      pltpu.sync_copy(dst_h.at[pl.ds(t * ROWS_PT, ROWS_PT)], dst_v)

      def fill_ones(j, c):
        ones_v[pl.ds(j * 16, 16)] = jnp.full((16,), 1.0, jnp.float32)
        return c
      lax.fori_loop(0, ECHUNK // 16, fill_ones, 0)

      def fill_dv(j, c):
        dv_v[pl.ds(j * 16, 16)] = jnp.zeros((16,), jnp.float32)
        return c
      lax.fori_loop(0, NODES_PT // 16, fill_dv, 0)

      pltpu.sync_copy(dv_v, degs_s.at[pl.ds(nbase, NODES_PT)])
      pltpu.sync_copy(dv_v, degd_s.at[pl.ds(nbase, NODES_PT)])
      plsc.subcore_barrier()

      def degbody(c, carry):
        pltpu.sync_copy(ones_v, degs_s.at[src_v.at[c]], add=True)
        pltpu.sync_copy(ones_v, degd_s.at[dst_v.at[c]], add=True)
        return carry
      lax.fori_loop(0, ROWS_PT, degbody, 0)

      plsc.subcore_barrier()
      pltpu.sync_copy(degs_s.at[pl.ds(nbase, NODES_PT)],
                      deg_h.at[0, pl.ds(nbase, NODES_PT)])
      pltpu.sync_copy(degd_s.at[pl.ds(nbase, NODES_PT)],
                      deg_h.at[1, pl.ds(nbase, NODES_PT)])

  mesh = plsc.VectorSubcoreMesh(core_axis_name="c", subcore_axis_name="s")
  f = pl.kernel(
      body,
      out_type=[jax.ShapeDtypeStruct((2, NPAD), jnp.float32)],
      mesh=mesh,
      compiler_params=pltpu.CompilerParams(use_tc_tiling_on_sc=False),
      scratch_types=[
          pltpu.VMEM_SHARED((NPAD,), jnp.float32),
          pltpu.VMEM_SHARED((NPAD,), jnp.float32),
          pltpu.VMEM((ROWS_PT, ECHUNK), jnp.int32),
          pltpu.VMEM((ROWS_PT, ECHUNK), jnp.int32),
          pltpu.VMEM((ECHUNK,), jnp.float32),
          pltpu.VMEM((NODES_PT,), jnp.float32),
      ],
  )
  return f(src2d, dst2d)[0]


def _tc_fg(deg):
  def body(d_ref, o_ref):
    o_ref[...] = lax.rsqrt(jnp.maximum(d_ref[...], 1.0))

  return pl.pallas_call(
      body,
      out_shape=jax.ShapeDtypeStruct((2, NPAD), jnp.float32),
  )(deg)


def _recurrence_split(fdim2):
  def body(src_h, dst_h, x0_h, fg_h, tstack_h, tprime_h,
           accum_s, src_v, dst_v, f_v, g_v, buf_v, gsem, ssem):
    cid = lax.axis_index("c")
    t = lax.axis_index("s")
    nbase = t * NODES_PT
    pltpu.sync_copy(src_h.at[pl.ds(t * ROWS_PT, ROWS_PT)], src_v)
    pltpu.sync_copy(dst_h.at[pl.ds(t * ROWS_PT, ROWS_PT)], dst_v)
    pltpu.sync_copy(fg_h.at[0, pl.ds(nbase, NODES_PT)], f_v)
    pltpu.sync_copy(fg_h.at[1, pl.ds(nbase, NODES_PT)], g_v)
    my_x0 = x0_h.at[cid]
    my_ts = tstack_h.at[cid]
    my_tp = tprime_h.at[cid]

    def _sl(b):
      return pl.ds(nbase + b * NBLK, NBLK)

    def fill_zero(r, c):
      for fb in range(fdim2 // 16):
        buf_v[2, r, pl.ds(fb * 16, 16)] = jnp.zeros((16,), jnp.float32)
      return c
    lax.fori_loop(0, NBLK, fill_zero, 0)

    def zacc(b, c):
      pltpu.sync_copy(buf_v.at[2], accum_s.at[_sl(b)])
      return c
    lax.fori_loop(0, NBLKS_PT, zacc, 0)

    def initb(b, c):
      sl = _sl(b)
      pltpu.sync_copy(my_x0.at[sl], buf_v.at[0])
      pltpu.sync_copy(buf_v.at[0], my_ts.at[0, sl])

      def grp(jj, c2):
        f16 = f_v[pl.ds(b * NBLK + jj * 16, 16)]

        def lane(i2, c3):
          fs = _splat(f16, i2)
          r = jj * 16 + i2
          for fb in range(fdim2 // 16):
            row = buf_v[0, r, pl.ds(fb * 16, 16)]
            buf_v[1, r, pl.ds(fb * 16, 16)] = fs * row
          return c3
        lax.fori_loop(0, 16, lane, 0)
        return c2
      lax.fori_loop(0, NBLK // 16, grp, 0)
      pltpu.sync_copy(buf_v.at[1], my_tp.at[sl])
      return c
    lax.fori_loop(0, NBLKS_PT, initb, 0)

    plsc.subcore_barrier()

    def kstep(k, carry):
      _edge_phase(src_v, dst_v, my_tp, accum_s, buf_v, gsem, ssem)
      plsc.subcore_barrier()

      kidx = jnp.maximum(k - 2, 0)
      sp = jnp.where(k >= 2, 1.0, 0.0).astype(jnp.float32)
      ca = jnp.where(k >= 2, -2.0, -1.0).astype(jnp.float32)
      sp16 = jnp.full((16,), 1.0, jnp.float32) * sp

      lax.fori_loop(0, NBLK, fill_zero, 0)

      def nodeb(b, c):
        sl = _sl(b)
        pltpu.sync_copy(accum_s.at[sl], buf_v.at[0])
        pltpu.sync_copy(buf_v.at[2], accum_s.at[sl])
        pltpu.sync_copy(my_ts.at[kidx, sl], buf_v.at[1])

        def grp(jj, c2):
          g16 = g_v[pl.ds(b * NBLK + jj * 16, 16)] * ca
          f16 = f_v[pl.ds(b * NBLK + jj * 16, 16)]

          def lane(i2, c3):
            cg = _splat(g16, i2)
            fs = _splat(f16, i2)
            r = jj * 16 + i2
            for fb in range(fdim2 // 16):
              a = buf_v[0, r, pl.ds(fb * 16, 16)]
              tp = buf_v[1, r, pl.ds(fb * 16, 16)]
              tn = cg * a - sp16 * tp
              buf_v[1, r, pl.ds(fb * 16, 16)] = tn
              buf_v[0, r, pl.ds(fb * 16, 16)] = fs * tn
            return c3
          lax.fori_loop(0, 16, lane, 0)
          return c2
        lax.fori_loop(0, NBLK // 16, grp, 0)

        pltpu.sync_copy(buf_v.at[1], my_ts.at[k, sl])
        pltpu.sync_copy(buf_v.at[0], my_tp.at[sl])
        return c
      lax.fori_loop(0, NBLKS_PT, nodeb, 0)
      plsc.subcore_barrier()
      return carry

    lax.fori_loop(1, K, kstep, 0)

  mesh = plsc.VectorSubcoreMesh(core_axis_name="c", subcore_axis_name="s")
  return pl.kernel(
      body,
      out_type=[
          jax.ShapeDtypeStruct((2, K, NPAD, fdim2), jnp.float32),
          jax.ShapeDtypeStruct((2, NPAD, fdim2), jnp.float32),
      ],
      mesh=mesh,
      compiler_params=pltpu.CompilerParams(use_tc_tiling_on_sc=False),
      scratch_types=[
          pltpu.VMEM_SHARED((NPAD, fdim2), jnp.float32),
          pltpu.VMEM((ROWS_PT, ECHUNK), jnp.int32),
          pltpu.VMEM((ROWS_PT, ECHUNK), jnp.int32),
          pltpu.VMEM((NODES_PT,), jnp.float32),
          pltpu.VMEM((NODES_PT,), jnp.float32),
          pltpu.VMEM((NB, ECHUNK, fdim2), jnp.float32),
          pltpu.SemaphoreType.DMA((NB,)),
          pltpu.SemaphoreType.DMA((NB,)),
      ],
  )


def _recurrence(fdim, with_out):
  def body(src_h, dst_h, x0_h, fg_h, th_h, out_h, tstack_h, tprime_h,
           accum_s, src_v, dst_v, f_v, g_v, th_v, buf_v, gsem, ssem):
    cid = lax.axis_index("c")
    t = lax.axis_index("s")

    @pl.when(cid == 0)
    def _core0():
      nbase = t * NODES_PT
      pltpu.sync_copy(src_h.at[pl.ds(t * ROWS_PT, ROWS_PT)], src_v)
      pltpu.sync_copy(dst_h.at[pl.ds(t * ROWS_PT, ROWS_PT)], dst_v)
      pltpu.sync_copy(fg_h.at[0, pl.ds(nbase, NODES_PT)], f_v)
      pltpu.sync_copy(fg_h.at[1, pl.ds(nbase, NODES_PT)], g_v)
      pltpu.sync_copy(th_h.at[0], th_v)
      th16 = th_v[...]
      th0 = _splat(th16, 0)

      def _sl(b):
        return pl.ds(nbase + b * NBLK, NBLK)

      def fill_zero(r, c):
        for fb in range(fdim // 16):
          buf_v[2, r, pl.ds(fb * 16, 16)] = jnp.zeros((16,), jnp.float32)
        return c
      lax.fori_loop(0, NBLK, fill_zero, 0)

      def zacc(b, c):
        pltpu.sync_copy(buf_v.at[2], accum_s.at[_sl(b)])
        return c
      lax.fori_loop(0, NBLKS_PT, zacc, 0)

      def initb(b, c):
        sl = _sl(b)
        pltpu.sync_copy(x0_h.at[sl], buf_v.at[0])
        pltpu.sync_copy(buf_v.at[0], tstack_h.at[0, sl])

        def grp(jj, c2):
          f16 = f_v[pl.ds(b * NBLK + jj * 16, 16)]

          def lane(i2, c3):
            fs = _splat(f16, i2)
            r = jj * 16 + i2
            for fb in range(fdim // 16):
              row = buf_v[0, r, pl.ds(fb * 16, 16)]
              buf_v[1, r, pl.ds(fb * 16, 16)] = fs * row
              if with_out:
                buf_v[3, r, pl.ds(fb * 16, 16)] = th0 * row
            return c3
          lax.fori_loop(0, 16, lane, 0)
          return c2
        lax.fori_loop(0, NBLK // 16, grp, 0)
        pltpu.sync_copy(buf_v.at[1], tprime_h.at[sl])
        if with_out:
          pltpu.sync_copy(buf_v.at[3], out_h.at[sl])
        return c
      lax.fori_loop(0, NBLKS_PT, initb, 0)

      plsc.subcore_barrier()

      def kstep(k, carry):
        _edge_phase(src_v, dst_v, tprime_h, accum_s, buf_v, gsem, ssem)
        plsc.subcore_barrier()

        kidx = jnp.maximum(k - 2, 0)
        sp = jnp.where(k >= 2, 1.0, 0.0).astype(jnp.float32)
        ca = jnp.where(k >= 2, -2.0, -1.0).astype(jnp.float32)
        sp16 = jnp.full((16,), 1.0, jnp.float32) * sp
        thk = _splat(th16, k)

        lax.fori_loop(0, NBLK, fill_zero, 0)

        def nodeb(b, c):
          sl = _sl(b)
          pltpu.sync_copy(accum_s.at[sl], buf_v.at[0])
          pltpu.sync_copy(buf_v.at[2], accum_s.at[sl])
          pltpu.sync_copy(tstack_h.at[kidx, sl], buf_v.at[1])
          if with_out:
            pltpu.sync_copy(out_h.at[sl], buf_v.at[3])

          def grp(jj, c2):
            g16 = g_v[pl.ds(b * NBLK + jj * 16, 16)] * ca
            f16 = f_v[pl.ds(b * NBLK + jj * 16, 16)]

            def lane(i2, c3):
              cg = _splat(g16, i2)
              fs = _splat(f16, i2)
              r = jj * 16 + i2
              for fb in range(fdim // 16):
                a = buf_v[0, r, pl.ds(fb * 16, 16)]
                tp = buf_v[1, r, pl.ds(fb * 16, 16)]
                tn = cg * a - sp16 * tp
                buf_v[1, r, pl.ds(fb * 16, 16)] = tn
                buf_v[0, r, pl.ds(fb * 16, 16)] = fs * tn
                if with_out:
                  buf_v[3, r, pl.ds(fb * 16, 16)] = (
                      buf_v[3, r, pl.ds(fb * 16, 16)] + thk * tn)
              return c3
            lax.fori_loop(0, 16, lane, 0)
            return c2
          lax.fori_loop(0, NBLK // 16, grp, 0)

          pltpu.sync_copy(buf_v.at[1], tstack_h.at[k, sl])
          pltpu.sync_copy(buf_v.at[0], tprime_h.at[sl])
          if with_out:
            pltpu.sync_copy(buf_v.at[3], out_h.at[sl])
          return c
        lax.fori_loop(0, NBLKS_PT, nodeb, 0)
        plsc.subcore_barrier()
        return carry

      lax.fori_loop(1, K, kstep, 0)

  mesh = plsc.VectorSubcoreMesh(core_axis_name="c", subcore_axis_name="s")
  return pl.kernel(
      body,
      out_type=[
          jax.ShapeDtypeStruct((NPAD, fdim), jnp.float32),
          jax.ShapeDtypeStruct((K, NPAD, fdim), jnp.float32),
          jax.ShapeDtypeStruct((NPAD, fdim), jnp.float32),
      ],
      mesh=mesh,
      compiler_params=pltpu.CompilerParams(use_tc_tiling_on_sc=False),
      scratch_types=[
          pltpu.VMEM_SHARED((NPAD, fdim), jnp.float32),
          pltpu.VMEM((ROWS_PT, ECHUNK), jnp.int32),
          pltpu.VMEM((ROWS_PT, ECHUNK), jnp.int32),
          pltpu.VMEM((NODES_PT,), jnp.float32),
          pltpu.VMEM((NODES_PT,), jnp.float32),
          pltpu.VMEM((16,), jnp.float32),
          pltpu.VMEM((NB, ECHUNK, fdim), jnp.float32),
          pltpu.SemaphoreType.DMA((NB,)),
          pltpu.SemaphoreType.DMA((NB,)),
      ],
  )


def _tc_matmul(x, w):
  blk = 1024

  def body(x_ref, w_ref, o_ref):
    o_ref[...] = jnp.dot(x_ref[...], w_ref[...],
                         preferred_element_type=jnp.float32)

  return pl.pallas_call(
      body,
      grid=(NPAD // blk,),
      in_specs=[
          pl.BlockSpec((blk, D), lambda i: (i, 0)),
          pl.BlockSpec((D, HID), lambda i: (0, 0)),
      ],
      out_specs=pl.BlockSpec((blk, HID), lambda i: (i, 0)),
      out_shape=jax.ShapeDtypeStruct((NPAD, HID), jnp.float32),
  )(x, w)


def _tc_combine(tstack, v):
  blk = 1024
  f2 = HID // 2

  def body(t_ref, v_ref, o_ref):
    acc = jnp.zeros((blk, CPAD), jnp.float32)
    for h in range(2):
      for k in range(K):
        acc = acc + jnp.dot(t_ref[h, k], v_ref[h, k],
                            preferred_element_type=jnp.float32)
    o_ref[...] = acc

  return pl.pallas_call(
      body,
      grid=(NPAD // blk,),
      in_specs=[
          pl.BlockSpec((2, K, blk, f2), lambda i: (0, 0, i, 0)),
          pl.BlockSpec((2, K, f2, CPAD), lambda i: (0, 0, 0, 0)),
      ],
      out_specs=pl.BlockSpec((blk, CPAD), lambda i: (i, 0)),
      out_shape=jax.ShapeDtypeStruct((NPAD, CPAD), jnp.float32),
  )(tstack, v)


def _tc_head(outpre):
  blk = 1024

  def body(x_ref, o_ref):
    x = x_ref[...]
    h = jnp.where(x > 0, x, jnp.exp(x) - 1.0)
    mask = lax.broadcasted_iota(jnp.int32, (blk, CPAD), 1) < CLS
    neg = jnp.float32(-1e30)
    hm = jnp.where(mask, h, neg)
    mx = jnp.max(hm, axis=1, keepdims=True)
    ex = jnp.where(mask, jnp.exp(h - mx), 0.0)
    lse = jnp.log(jnp.sum(ex, axis=1, keepdims=True))
    o_ref[...] = h - mx - lse

  return pl.pallas_call(
      body,
      grid=(NPAD // blk,),
      in_specs=[pl.BlockSpec((blk, CPAD), lambda i: (i, 0))],
      out_specs=pl.BlockSpec((blk, CPAD), lambda i: (i, 0)),
      out_shape=jax.ShapeDtypeStruct((NPAD, CPAD), jnp.float32),
  )(outpre)


@jax.jit
def kernel(x, edge_index, W1, theta1, W2, theta2):
  E = edge_index.shape[1]
  dummy = NPAD - 1
  src = jnp.concatenate(
      [edge_index[0], jnp.full((EPAD - E,), dummy, jnp.int32)])
  dst = jnp.concatenate(
      [edge_index[1], jnp.full((EPAD - E,), dummy, jnp.int32)])
  src2d = src.reshape(EPAD // ECHUNK, ECHUNK)
  dst2d = dst.reshape(EPAD // ECHUNK, ECHUNK)

  xpad = jnp.pad(x, ((0, NPAD - N), (0, 0)))
  v = jnp.einsum("hk,hfc->kfc", theta1, W2.reshape(HEADS, HID, CLS))
  vpad = jnp.pad(v, ((0, 0), (0, 0), (0, CPAD - CLS)))
  th_dummy = jnp.zeros((1, K), jnp.float32)
  theta2p = theta2.astype(jnp.float32)

  xw1 = _tc_matmul(xpad, W1)
  deg = _sc_degrees(src2d, dst2d)
  fg = _tc_fg(deg)
  f2 = HID // 2
  xw1s = jnp.stack([xw1[:, :f2], xw1[:, f2:]])
  vs = jnp.stack([vpad[:, :f2, :], vpad[:, f2:, :]])
  tstack, _ = _recurrence_split(f2)(src2d, dst2d, xw1s, fg)
  xw2 = _tc_combine(tstack, vs)
  outpre, _, _ = _recurrence(CPAD, True)(
      src2d, dst2d, xw2, fg, theta2p)
  out = _tc_head(outpre)
  return out[:N, :CLS]


# speedup vs baseline: 12.0239x; 1.0366x over previous
import functools

import jax
import jax.numpy as jnp
from jax import lax
from jax.experimental import pallas as pl
from jax.experimental.pallas import tpu as pltpu
from jax.experimental.pallas import tpu_sc as plsc

N = 10000
D = 128
HID = 64
HEADS = 4
CLS = 7
K = 16

NPAD = 10240
EPAD = 327680
CPAD = 16
TILES = 16
NODES_PT = NPAD // TILES
ECHUNK = 128
ROWS_PT = EPAD // ECHUNK // TILES
NBLK = 128
NBLKS_PT = NODES_PT // NBLK
NB = 12
NG = NB // 2


def _splat(v16, i):
  idx = jnp.full((16,), i, jnp.int32)
  return v16.at[idx].get(mode="promise_in_bounds")


def _edge_phase(src_v, dst_v, tprime_h, accum_s, buf_v, gsem, ssem):
  def _gather(c):
    s = c % NB
    return pltpu.make_async_copy(
        tprime_h.at[src_v.at[c]], buf_v.at[s], gsem.at[s])

  def _scatter(c):
    s = c % NB
    return pltpu.make_async_copy(
        buf_v.at[s], accum_s.at[dst_v.at[c]], ssem.at[s])

  def prolog(j, carry):
    _gather(j).start()
    return carry
  lax.fori_loop(0, NG, prolog, 0)

  def body(c, carry):
    _gather(c).wait()
    _scatter(c).start(add=True)

    @pl.when(c + NG < ROWS_PT)
    def _():
      @pl.when(c >= NB - NG)
      def _():
        _scatter(c - (NB - NG)).wait()
      _gather(c + NG).start()
    return carry
  lax.fori_loop(0, ROWS_PT, body, 0)

  def drain(x, carry):
    _scatter(x).wait()
    return carry
  lax.fori_loop(ROWS_PT - NB, ROWS_PT, drain, 0)


def _sc_degrees(src2d, dst2d):
  def body(src_h, dst_h, deg_h, degs_s, degd_s, src_v, dst_v, ones_v, dv_v):
    cid = lax.axis_index("c")
    t = lax.axis_index("s")

    @pl.when(cid == 0)
    def _core0():
      nbase = t * NODES_PT
      pltpu.sync_copy(src_h.at[pl.ds(t * ROWS_PT, ROWS_PT)], src_v)
      pltpu.sync_copy(dst_h.at[pl.ds(t * ROWS_PT, ROWS_PT)], dst_v)

      def fill_ones(j, c):
        ones_v[pl.ds(j * 16, 16)] = jnp.full((16,), 1.0, jnp.float32)
        return c
      lax.fori_loop(0, ECHUNK // 16, fill_ones, 0)

      def fill_dv(j, c):
        dv_v[pl.ds(j * 16, 16)] = jnp.zeros((16,), jnp.float32)
        return c
      lax.fori_loop(0, NODES_PT // 16, fill_dv, 0)

      pltpu.sync_copy(dv_v, degs_s.at[pl.ds(nbase, NODES_PT)])
      pltpu.sync_copy(dv_v, degd_s.at[pl.ds(nbase, NODES_PT)])
      plsc.subcore_barrier()

      def degbody(c, carry):
        pltpu.sync_copy(ones_v, degs_s.at[src_v.at[c]], add=True)
        pltpu.sync_copy(ones_v, degd_s.at[dst_v.at[c]], add=True)
        return carry
      lax.fori_loop(0, ROWS_PT, degbody, 0)

      plsc.subcore_barrier()
      pltpu.sync_copy(degs_s.at[pl.ds(nbase, NODES_PT)],
                      deg_h.at[0, pl.ds(nbase, NODES_PT)])
      pltpu.sync_copy(degd_s.at[pl.ds(nbase, NODES_PT)],
                      deg_h.at[1, pl.ds(nbase, NODES_PT)])

  mesh = plsc.VectorSubcoreMesh(core_axis_name="c", subcore_axis_name="s")
  f = pl.kernel(
      body,
      out_type=[jax.ShapeDtypeStruct((2, NPAD), jnp.float32)],
      mesh=mesh,
      compiler_params=pltpu.CompilerParams(use_tc_tiling_on_sc=False),
      scratch_types=[
          pltpu.VMEM_SHARED((NPAD,), jnp.float32),
          pltpu.VMEM_SHARED((NPAD,), jnp.float32),
          pltpu.VMEM((ROWS_PT, ECHUNK), jnp.int32),
          pltpu.VMEM((ROWS_PT, ECHUNK), jnp.int32),
          pltpu.VMEM((ECHUNK,), jnp.float32),
          pltpu.VMEM((NODES_PT,), jnp.float32),
      ],
  )
  return f(src2d, dst2d)[0]


def _tc_fg(deg):
  def body(d_ref, o_ref):
    o_ref[...] = lax.rsqrt(jnp.maximum(d_ref[...], 1.0))

  return pl.pallas_call(
      body,
      out_shape=jax.ShapeDtypeStruct((2, NPAD), jnp.float32),
  )(deg)


def _recurrence_split(fdim2):
  def body(src_h, dst_h, x0_h, fg_h, tstack_h, tprime_h,
           accum_s, src_v, dst_v, f_v, g_v, buf_v, gsem, ssem):
    cid = lax.axis_index("c")
    t = lax.axis_index("s")
    nbase = t * NODES_PT
    pltpu.sync_copy(src_h.at[pl.ds(t * ROWS_PT, ROWS_PT)], src_v)
    pltpu.sync_copy(dst_h.at[pl.ds(t * ROWS_PT, ROWS_PT)], dst_v)
    pltpu.sync_copy(fg_h.at[0, pl.ds(nbase, NODES_PT)], f_v)
    pltpu.sync_copy(fg_h.at[1, pl.ds(nbase, NODES_PT)], g_v)
    my_x0 = x0_h.at[cid]
    my_ts = tstack_h.at[cid]
    my_tp = tprime_h.at[cid]

    def _sl(b):
      return pl.ds(nbase + b * NBLK, NBLK)

    def fill_zero(r, c):
      for fb in range(fdim2 // 16):
        buf_v[2, r, pl.ds(fb * 16, 16)] = jnp.zeros((16,), jnp.float32)
      return c
    lax.fori_loop(0, NBLK, fill_zero, 0)

    def zacc(b, c):
      pltpu.sync_copy(buf_v.at[2], accum_s.at[_sl(b)])
      return c
    lax.fori_loop(0, NBLKS_PT, zacc, 0)

    def initb(b, c):
      sl = _sl(b)
      pltpu.sync_copy(my_x0.at[sl], buf_v.at[0])
      pltpu.sync_copy(buf_v.at[0], my_ts.at[0, sl])

      def grp(jj, c2):
        f16 = f_v[pl.ds(b * NBLK + jj * 16, 16)]

        def lane(i2, c3):
          fs = _splat(f16, i2)
          r = jj * 16 + i2
          for fb in range(fdim2 // 16):
            row = buf_v[0, r, pl.ds(fb * 16, 16)]
            buf_v[1, r, pl.ds(fb * 16, 16)] = fs * row
          return c3
        lax.fori_loop(0, 16, lane, 0)
        return c2
      lax.fori_loop(0, NBLK // 16, grp, 0)
      pltpu.sync_copy(buf_v.at[1], my_tp.at[sl])
      return c
    lax.fori_loop(0, NBLKS_PT, initb, 0)

    plsc.subcore_barrier()

    def kstep(k, carry):
      _edge_phase(src_v, dst_v, my_tp, accum_s, buf_v, gsem, ssem)
      plsc.subcore_barrier()

      kidx = jnp.maximum(k - 2, 0)
      sp = jnp.where(k >= 2, 1.0, 0.0).astype(jnp.float32)
      ca = jnp.where(k >= 2, -2.0, -1.0).astype(jnp.float32)
      sp16 = jnp.full((16,), 1.0, jnp.float32) * sp

      lax.fori_loop(0, NBLK, fill_zero, 0)

      def nodeb(b, c):
        sl = _sl(b)
        pltpu.sync_copy(accum_s.at[sl], buf_v.at[0])
        pltpu.sync_copy(buf_v.at[2], accum_s.at[sl])
        pltpu.sync_copy(my_ts.at[kidx, sl], buf_v.at[1])

        def grp(jj, c2):
          g16 = g_v[pl.ds(b * NBLK + jj * 16, 16)] * ca
          f16 = f_v[pl.ds(b * NBLK + jj * 16, 16)]

          def lane(i2, c3):
            cg = _splat(g16, i2)
            fs = _splat(f16, i2)
            r = jj * 16 + i2
            for fb in range(fdim2 // 16):
              a = buf_v[0, r, pl.ds(fb * 16, 16)]
              tp = buf_v[1, r, pl.ds(fb * 16, 16)]
              tn = cg * a - sp16 * tp
              buf_v[1, r, pl.ds(fb * 16, 16)] = tn
              buf_v[0, r, pl.ds(fb * 16, 16)] = fs * tn
            return c3
          lax.fori_loop(0, 16, lane, 0)
          return c2
        lax.fori_loop(0, NBLK // 16, grp, 0)

        pltpu.sync_copy(buf_v.at[1], my_ts.at[k, sl])
        pltpu.sync_copy(buf_v.at[0], my_tp.at[sl])
        return c
      lax.fori_loop(0, NBLKS_PT, nodeb, 0)
      plsc.subcore_barrier()
      return carry

    lax.fori_loop(1, K, kstep, 0)

  mesh = plsc.VectorSubcoreMesh(core_axis_name="c", subcore_axis_name="s")
  return pl.kernel(
      body,
      out_type=[
          jax.ShapeDtypeStruct((2, K, NPAD, fdim2), jnp.float32),
          jax.ShapeDtypeStruct((2, NPAD, fdim2), jnp.float32),
      ],
      mesh=mesh,
      compiler_params=pltpu.CompilerParams(use_tc_tiling_on_sc=False),
      scratch_types=[
          pltpu.VMEM_SHARED((NPAD, fdim2), jnp.float32),
          pltpu.VMEM((ROWS_PT, ECHUNK), jnp.int32),
          pltpu.VMEM((ROWS_PT, ECHUNK), jnp.int32),
          pltpu.VMEM((NODES_PT,), jnp.float32),
          pltpu.VMEM((NODES_PT,), jnp.float32),
          pltpu.VMEM((NB, ECHUNK, fdim2), jnp.float32),
          pltpu.SemaphoreType.DMA((NB,)),
          pltpu.SemaphoreType.DMA((NB,)),
      ],
  )


def _recurrence(fdim, with_out):
  def body(src_h, dst_h, x0_h, fg_h, th_h, out_h, tstack_h, tprime_h,
           accum_s, src_v, dst_v, f_v, g_v, th_v, buf_v, gsem, ssem):
    cid = lax.axis_index("c")
    t = lax.axis_index("s")

    @pl.when(cid == 0)
    def _core0():
      nbase = t * NODES_PT
      pltpu.sync_copy(src_h.at[pl.ds(t * ROWS_PT, ROWS_PT)], src_v)
      pltpu.sync_copy(dst_h.at[pl.ds(t * ROWS_PT, ROWS_PT)], dst_v)
      pltpu.sync_copy(fg_h.at[0, pl.ds(nbase, NODES_PT)], f_v)
      pltpu.sync_copy(fg_h.at[1, pl.ds(nbase, NODES_PT)], g_v)
      pltpu.sync_copy(th_h.at[0], th_v)
      th16 = th_v[...]
      th0 = _splat(th16, 0)

      def _sl(b):
        return pl.ds(nbase + b * NBLK, NBLK)

      def fill_zero(r, c):
        for fb in range(fdim // 16):
          buf_v[2, r, pl.ds(fb * 16, 16)] = jnp.zeros((16,), jnp.float32)
        return c
      lax.fori_loop(0, NBLK, fill_zero, 0)

      def zacc(b, c):
        pltpu.sync_copy(buf_v.at[2], accum_s.at[_sl(b)])
        return c
      lax.fori_loop(0, NBLKS_PT, zacc, 0)

      def initb(b, c):
        sl = _sl(b)
        pltpu.sync_copy(x0_h.at[sl], buf_v.at[0])
        pltpu.sync_copy(buf_v.at[0], tstack_h.at[0, sl])

        def grp(jj, c2):
          f16 = f_v[pl.ds(b * NBLK + jj * 16, 16)]

          def lane(i2, c3):
            fs = _splat(f16, i2)
            r = jj * 16 + i2
            for fb in range(fdim // 16):
              row = buf_v[0, r, pl.ds(fb * 16, 16)]
              buf_v[1, r, pl.ds(fb * 16, 16)] = fs * row
              if with_out:
                buf_v[3, r, pl.ds(fb * 16, 16)] = th0 * row
            return c3
          lax.fori_loop(0, 16, lane, 0)
          return c2
        lax.fori_loop(0, NBLK // 16, grp, 0)
        pltpu.sync_copy(buf_v.at[1], tprime_h.at[sl])
        if with_out:
          pltpu.sync_copy(buf_v.at[3], out_h.at[sl])
        return c
      lax.fori_loop(0, NBLKS_PT, initb, 0)

      plsc.subcore_barrier()

      def kstep(k, carry):
        _edge_phase(src_v, dst_v, tprime_h, accum_s, buf_v, gsem, ssem)
        plsc.subcore_barrier()

        kidx = jnp.maximum(k - 2, 0)
        sp = jnp.where(k >= 2, 1.0, 0.0).astype(jnp.float32)
        ca = jnp.where(k >= 2, -2.0, -1.0).astype(jnp.float32)
        sp16 = jnp.full((16,), 1.0, jnp.float32) * sp
        thk = _splat(th16, k)

        lax.fori_loop(0, NBLK, fill_zero, 0)

        def nodeb(b, c):
          sl = _sl(b)
          pltpu.sync_copy(accum_s.at[sl], buf_v.at[0])
          pltpu.sync_copy(buf_v.at[2], accum_s.at[sl])
          pltpu.sync_copy(tstack_h.at[kidx, sl], buf_v.at[1])
          if with_out:
            pltpu.sync_copy(out_h.at[sl], buf_v.at[3])

          def grp(jj, c2):
            g16 = g_v[pl.ds(b * NBLK + jj * 16, 16)] * ca
            f16 = f_v[pl.ds(b * NBLK + jj * 16, 16)]

            def lane(i2, c3):
              cg = _splat(g16, i2)
              fs = _splat(f16, i2)
              r = jj * 16 + i2
              for fb in range(fdim // 16):
                a = buf_v[0, r, pl.ds(fb * 16, 16)]
                tp = buf_v[1, r, pl.ds(fb * 16, 16)]
                tn = cg * a - sp16 * tp
                buf_v[1, r, pl.ds(fb * 16, 16)] = tn
                buf_v[0, r, pl.ds(fb * 16, 16)] = fs * tn
                if with_out:
                  buf_v[3, r, pl.ds(fb * 16, 16)] = (
                      buf_v[3, r, pl.ds(fb * 16, 16)] + thk * tn)
              return c3
            lax.fori_loop(0, 16, lane, 0)
            return c2
          lax.fori_loop(0, NBLK // 16, grp, 0)

          pltpu.sync_copy(buf_v.at[1], tstack_h.at[k, sl])
          pltpu.sync_copy(buf_v.at[0], tprime_h.at[sl])
          if with_out:
            pltpu.sync_copy(buf_v.at[3], out_h.at[sl])
          return c
        lax.fori_loop(0, NBLKS_PT, nodeb, 0)
        plsc.subcore_barrier()
        return carry

      lax.fori_loop(1, K, kstep, 0)

  mesh = plsc.VectorSubcoreMesh(core_axis_name="c", subcore_axis_name="s")
  return pl.kernel(
      body,
      out_type=[
          jax.ShapeDtypeStruct((NPAD, fdim), jnp.float32),
          jax.ShapeDtypeStruct((K, NPAD, fdim), jnp.float32),
          jax.ShapeDtypeStruct((NPAD, fdim), jnp.float32),
      ],
      mesh=mesh,
      compiler_params=pltpu.CompilerParams(use_tc_tiling_on_sc=False),
      scratch_types=[
          pltpu.VMEM_SHARED((NPAD, fdim), jnp.float32),
          pltpu.VMEM((ROWS_PT, ECHUNK), jnp.int32),
          pltpu.VMEM((ROWS_PT, ECHUNK), jnp.int32),
          pltpu.VMEM((NODES_PT,), jnp.float32),
          pltpu.VMEM((NODES_PT,), jnp.float32),
          pltpu.VMEM((16,), jnp.float32),
          pltpu.VMEM((NB, ECHUNK, fdim), jnp.float32),
          pltpu.SemaphoreType.DMA((NB,)),
          pltpu.SemaphoreType.DMA((NB,)),
      ],
  )


def _tc_matmul(x, w):
  blk = 1024

  def body(x_ref, w_ref, o_ref):
    o_ref[...] = jnp.dot(x_ref[...], w_ref[...],
                         preferred_element_type=jnp.float32)

  return pl.pallas_call(
      body,
      grid=(NPAD // blk,),
      in_specs=[
          pl.BlockSpec((blk, D), lambda i: (i, 0)),
          pl.BlockSpec((D, HID), lambda i: (0, 0)),
      ],
      out_specs=pl.BlockSpec((blk, HID), lambda i: (i, 0)),
      out_shape=jax.ShapeDtypeStruct((NPAD, HID), jnp.float32),
  )(x, w)


def _tc_combine(tstack, v):
  blk = 1024
  f2 = HID // 2

  def body(t_ref, v_ref, o_ref):
    acc = jnp.zeros((blk, CPAD), jnp.float32)
    for h in range(2):
      for k in range(K):
        acc = acc + jnp.dot(t_ref[h, k], v_ref[h, k],
                            preferred_element_type=jnp.float32)
    o_ref[...] = acc

  return pl.pallas_call(
      body,
      grid=(NPAD // blk,),
      in_specs=[
          pl.BlockSpec((2, K, blk, f2), lambda i: (0, 0, i, 0)),
          pl.BlockSpec((2, K, f2, CPAD), lambda i: (0, 0, 0, 0)),
      ],
      out_specs=pl.BlockSpec((blk, CPAD), lambda i: (i, 0)),
      out_shape=jax.ShapeDtypeStruct((NPAD, CPAD), jnp.float32),
  )(tstack, v)


def _tc_head(outpre):
  blk = 1024

  def body(x_ref, o_ref):
    x = x_ref[...]
    h = jnp.where(x > 0, x, jnp.exp(x) - 1.0)
    mask = lax.broadcasted_iota(jnp.int32, (blk, CPAD), 1) < CLS
    neg = jnp.float32(-1e30)
    hm = jnp.where(mask, h, neg)
    mx = jnp.max(hm, axis=1, keepdims=True)
    ex = jnp.where(mask, jnp.exp(h - mx), 0.0)
    lse = jnp.log(jnp.sum(ex, axis=1, keepdims=True))
    o_ref[...] = h - mx - lse

  return pl.pallas_call(
      body,
      grid=(NPAD // blk,),
      in_specs=[pl.BlockSpec((blk, CPAD), lambda i: (i, 0))],
      out_specs=pl.BlockSpec((blk, CPAD), lambda i: (i, 0)),
      out_shape=jax.ShapeDtypeStruct((NPAD, CPAD), jnp.float32),
  )(outpre)


@jax.jit
def kernel(x, edge_index, W1, theta1, W2, theta2):
  E = edge_index.shape[1]
  dummy = NPAD - 1
  src = jnp.concatenate(
      [edge_index[0], jnp.full((EPAD - E,), dummy, jnp.int32)])
  dst = jnp.concatenate(
      [edge_index[1], jnp.full((EPAD - E,), dummy, jnp.int32)])
  src2d = src.reshape(EPAD // ECHUNK, ECHUNK)
  dst2d = dst.reshape(EPAD // ECHUNK, ECHUNK)

  xpad = jnp.pad(x, ((0, NPAD - N), (0, 0)))
  v = jnp.einsum("hk,hfc->kfc", theta1, W2.reshape(HEADS, HID, CLS))
  vpad = jnp.pad(v, ((0, 0), (0, 0), (0, CPAD - CLS)))
  th_dummy = jnp.zeros((1, K), jnp.float32)
  theta2p = theta2.astype(jnp.float32)

  xw1 = _tc_matmul(xpad, W1)
  deg = _sc_degrees(src2d, dst2d)
  fg = _tc_fg(deg)
  f2 = HID // 2
  xw1s = jnp.stack([xw1[:, :f2], xw1[:, f2:]])
  vs = jnp.stack([vpad[:, :f2, :], vpad[:, f2:, :]])
  tstack, _ = _recurrence_split(f2)(src2d, dst2d, xw1s, fg)
  xw2 = _tc_combine(tstack, vs)
  outpre, _, _ = _recurrence(CPAD, True)(
      src2d, dst2d, xw2, fg, theta2p)
  out = _tc_head(outpre)
  return out[:N, :CLS]
